# baseline (speedup 1.0000x reference)
import jax
import jax.numpy as jnp
from jax import lax
from jax.experimental import pallas as pl
from jax.experimental.pallas import tpu as pltpu
from jax.experimental.pallas import tpu_sc as plsc

N_NODES = 10000
NPAD = 10240
N_EDGES = 320000
D = 128
DH = D // 2
C = 16

NC = 2
NS = 16
NW = NC * NS
CHUNK = 128

EPT = N_EDGES // NS
NCH_F = 158
EPT_PAD = NCH_F * CHUNK

EPW = N_EDGES // NW
NCH_E = 80
EPW_PAD = NCH_E * CHUNK

RPT = NPAD // NS

BR = 2048

_MESH = plsc.VectorSubcoreMesh(core_axis_name="c", subcore_axis_name="s",
                               num_cores=NC, num_subcores=NS)


NBUF = 4


def _edge_loop_sync(table, src_v, dst_v, acc, bufs, gsem, nchunk, deg=None,
                    deg_parity=0):
    def chunk(j, b):
        pltpu.make_async_copy(table.at[src_v.at[j]], bufs[b], gsem[b]).wait()
        pltpu.sync_copy(bufs[b], acc.at[dst_v.at[j]], add=True)
        if deg is not None and b % 2 == deg_parity:
            pltpu.sync_copy(deg[0], deg[1].at[dst_v.at[j]], add=True)

        @pl.when(j + 2 < nchunk)
        def _():
            pltpu.make_async_copy(table.at[src_v.at[j + 2]], bufs[b],
                                  gsem[b]).start()

    pltpu.make_async_copy(table.at[src_v.at[0]], bufs[0], gsem[0]).start()
    pltpu.make_async_copy(table.at[src_v.at[1]], bufs[1], gsem[1]).start()

    def step(k, carry):
        chunk(2 * k, 0)
        chunk(2 * k + 1, 1)
        return carry

    lax.fori_loop(0, nchunk // 2, step, 0)


def _edge_loop(table, src_v, dst_v, acc, bufs, gsem, ssem, nchunk, deg=None):
    pltpu.make_async_copy(table.at[src_v.at[0]], bufs[0], gsem[0]).start()
    pltpu.make_async_copy(table.at[src_v.at[1]], bufs[1], gsem[1]).start()

    def step(k, carry):
        for b in range(NBUF):
            j = NBUF * k + b
            bn = (b + 2) % NBUF
            pltpu.make_async_copy(table.at[src_v.at[j]], bufs[b], gsem[b]).wait()
            pltpu.async_copy(bufs[b], acc.at[dst_v.at[j]], ssem[b], add=True)
            if deg is not None:
                pltpu.async_copy(deg[0], deg[1].at[dst_v.at[j]], deg[2][b],
                                 add=True)

            @pl.when(j >= 2)
            def _():
                pltpu.make_async_copy(bufs[bn], acc.at[dst_v.at[j - 2]],
                                      ssem[bn]).wait()
                if deg is not None:
                    pltpu.make_async_copy(deg[0], deg[1].at[dst_v.at[j - 2]],
                                          deg[2][bn]).wait()

            @pl.when(j + 2 < nchunk)
            def _():
                pltpu.make_async_copy(table.at[src_v.at[j + 2]], bufs[bn],
                                      gsem[bn]).start()
        return carry

    lax.fori_loop(0, nchunk // NBUF, step, 0)
    for b in (2, 3):
        j = nchunk - NBUF + b
        pltpu.make_async_copy(bufs[b], acc.at[dst_v.at[j]], ssem[b]).wait()
        if deg is not None:
            pltpu.make_async_copy(deg[0], deg[1].at[dst_v.at[j]],
                                  deg[2][b]).wait()


def _make_sc_agg_split(with_deg):
    out_type = [jax.ShapeDtypeStruct((NC, NPAD, DH), jnp.float32)]
    scratch = [
        pltpu.VMEM((NCH_F, CHUNK), jnp.int32),
        pltpu.VMEM((NCH_F, CHUNK), jnp.int32),
        pltpu.VMEM((CHUNK, DH), jnp.float32),
        pltpu.VMEM((CHUNK, DH), jnp.float32),
        pltpu.VMEM_SHARED((NPAD, DH), jnp.float32),
        pltpu.SemaphoreType.DMA,
        pltpu.SemaphoreType.DMA,
    ]
    if with_deg:
        out_type.append(jax.ShapeDtypeStruct((NC, NPAD, 16), jnp.float32))
        scratch += [
            pltpu.VMEM((CHUNK, 16), jnp.float32),
            pltpu.VMEM_SHARED((NPAD, 16), jnp.float32),
        ]

    def body(p, srcs, dsts, zeros_d, *rest):
        if with_deg:
            (zeros16, ones_h, agg_out, deg_out,
             src_v, dst_v, buf0, buf1, acc, sem0, sem1, ones_v, dacc) = rest
        else:
            agg_out, src_v, dst_v, buf0, buf1, acc, sem0, sem1 = rest
        bufs = (buf0, buf1)
        gsem = (sem0, sem1)
        c = lax.axis_index("c")
        s = lax.axis_index("s")
        pltpu.sync_copy(srcs.at[s], src_v)
        pltpu.sync_copy(dsts.at[s], dst_v)
        pltpu.sync_copy(zeros_d.at[pl.ds(s * RPT, RPT)],
                        acc.at[pl.ds(s * RPT, RPT)])
        if with_deg:
            pltpu.sync_copy(zeros16.at[pl.ds(s * RPT, RPT)],
                            dacc.at[pl.ds(s * RPT, RPT)])
            pltpu.sync_copy(ones_h, ones_v)
        plsc.subcore_barrier()

        deg_args = (ones_v, dacc) if with_deg else None

        @pl.when(c == 0)
        def _():
            _edge_loop_sync(p.at[0], src_v, dst_v, acc, bufs, gsem, NCH_F,
                            deg=deg_args, deg_parity=0)

        @pl.when(c == 1)
        def _():
            _edge_loop_sync(p.at[1], src_v, dst_v, acc, bufs, gsem, NCH_F,
                            deg=deg_args, deg_parity=1)

        plsc.subcore_barrier()
        pltpu.sync_copy(acc.at[pl.ds(s * RPT, RPT)],
                        agg_out.at[c, pl.ds(s * RPT, RPT)])
        if with_deg:
            pltpu.sync_copy(dacc.at[pl.ds(s * RPT, RPT)],
                            deg_out.at[c, pl.ds(s * RPT, RPT)])

    out_spec = tuple(out_type) if len(out_type) > 1 else out_type[0]
    return pl.kernel(body, out_type=out_spec, mesh=_MESH,
                     scratch_types=tuple(scratch),
                     compiler_params=pltpu.CompilerParams(
                         use_tc_tiling_on_sc=False))


def _make_sc_agg_edges(d):
    scratch = [
        pltpu.VMEM((NCH_E, CHUNK), jnp.int32),
        pltpu.VMEM((NCH_E, CHUNK), jnp.int32),
        pltpu.VMEM((CHUNK, d), jnp.float32),
        pltpu.VMEM((CHUNK, d), jnp.float32),
        pltpu.VMEM_SHARED((NPAD, d), jnp.float32),
        pltpu.SemaphoreType.DMA,
        pltpu.SemaphoreType.DMA,
    ]

    def body(p, srcs, dsts, zeros_d, agg_out,
             src_v, dst_v, buf0, buf1, acc, sem0, sem1):
        c = lax.axis_index("c")
        s = lax.axis_index("s")
        w = c * NS + s
        pltpu.sync_copy(srcs.at[w], src_v)
        pltpu.sync_copy(dsts.at[w], dst_v)
        pltpu.sync_copy(zeros_d.at[pl.ds(s * RPT, RPT)],
                        acc.at[pl.ds(s * RPT, RPT)])
        plsc.subcore_barrier()
        _edge_loop_sync(p, src_v, dst_v, acc, (buf0, buf1), (sem0, sem1),
                        NCH_E)
        plsc.subcore_barrier()
        pltpu.sync_copy(acc.at[pl.ds(s * RPT, RPT)],
                        agg_out.at[c, pl.ds(s * RPT, RPT)])

    return pl.kernel(body,
                     out_type=jax.ShapeDtypeStruct((NC, NPAD, d), jnp.float32),
                     mesh=_MESH, scratch_types=tuple(scratch),
                     compiler_params=pltpu.CompilerParams(
                         use_tc_tiling_on_sc=False))


_sc_agg_split_deg = _make_sc_agg_split(with_deg=True)
_sc_agg_split = _make_sc_agg_split(with_deg=False)
_sc_agg16 = _make_sc_agg_edges(C)


def _mm_split_body(x_ref, w_ref, o_ref):
    p = jnp.dot(x_ref[...], w_ref[...], preferred_element_type=jnp.float32)
    o_ref[0] = p[:, :DH]
    o_ref[1] = p[:, DH:]


def _matmul_split(x, w):
    n, k = x.shape
    return pl.pallas_call(
        _mm_split_body,
        grid=(n // BR,),
        in_specs=[pl.BlockSpec((BR, k), lambda i: (i, 0)),
                  pl.BlockSpec((k, D), lambda i: (0, 0))],
        out_specs=pl.BlockSpec((NC, BR, DH), lambda i: (0, i, 0)),
        out_shape=jax.ShapeDtypeStruct((NC, n, DH), jnp.float32),
    )(x, w)


def _mid_body(split_p, x_ref, agg_ref, deg_ref, ws_ref, b_ref, wn_ref,
              h_ref, p_ref):
    agg = jnp.concatenate([agg_ref[0], agg_ref[1]], axis=1)
    deg = deg_ref[0, :, 0:1] + deg_ref[1, :, 0:1]
    inv = 1.0 / jnp.maximum(deg, 1.0)
    z = (jnp.dot(x_ref[...], ws_ref[...], preferred_element_type=jnp.float32)
         + agg * inv + b_ref[...])
    alpha = 1.6732632423543772
    scale = 1.0507009873554805
    h = scale * jnp.where(z > 0, z, alpha * (jnp.exp(z) - 1.0))
    h_ref[...] = h
    p = jnp.dot(h, wn_ref[...], preferred_element_type=jnp.float32)
    if split_p:
        p_ref[0] = p[:, :DH]
        p_ref[1] = p[:, DH:]
    else:
        p_ref[...] = p


def _mid_layer(x, agg, deg, w_self, b, w_neigh_next, split_p):
    n, k = x.shape
    m = w_neigh_next.shape[1]
    if split_p:
        p_spec = pl.BlockSpec((NC, BR, DH), lambda i: (0, i, 0))
        p_shape = jax.ShapeDtypeStruct((NC, n, DH), jnp.float32)
    else:
        p_spec = pl.BlockSpec((BR, m), lambda i: (i, 0))
        p_shape = jax.ShapeDtypeStruct((n, m), jnp.float32)
    body = lambda *refs: _mid_body(split_p, *refs)
    return pl.pallas_call(
        body,
        grid=(n // BR,),
        in_specs=[
            pl.BlockSpec((BR, k), lambda i: (i, 0)),
            pl.BlockSpec((NC, BR, DH), lambda i: (0, i, 0)),
            pl.BlockSpec((NC, BR, 16), lambda i: (0, i, 0)),
            pl.BlockSpec((k, k), lambda i: (0, 0)),
            pl.BlockSpec((1, k), lambda i: (0, 0)),
            pl.BlockSpec((k, m), lambda i: (0, 0)),
        ],
        out_specs=[pl.BlockSpec((BR, k), lambda i: (i, 0)), p_spec],
        out_shape=[jax.ShapeDtypeStruct((n, k), jnp.float32), p_shape],
    )(x, agg, deg, w_self, b, w_neigh_next)


def _last_body(h_ref, agg_ref, deg_ref, ws_ref, b_ref, o_ref):
    agg = agg_ref[0] + agg_ref[1]
    deg = deg_ref[0, :, 0:1] + deg_ref[1, :, 0:1]
    inv = 1.0 / jnp.maximum(deg, 1.0)
    z = (jnp.dot(h_ref[...], ws_ref[...], preferred_element_type=jnp.float32)
         + agg * inv + b_ref[...])
    z = z - jnp.max(z, axis=1, keepdims=True)
    e = jnp.exp(z)
    o_ref[...] = e / jnp.sum(e, axis=1, keepdims=True)


def _last_layer(h, agg, deg, w_self, b):
    n, k = h.shape
    m = w_self.shape[1]
    return pl.pallas_call(
        _last_body,
        grid=(n // BR,),
        in_specs=[
            pl.BlockSpec((BR, k), lambda i: (i, 0)),
            pl.BlockSpec((NC, BR, m), lambda i: (0, i, 0)),
            pl.BlockSpec((NC, BR, 16), lambda i: (0, i, 0)),
            pl.BlockSpec((k, m), lambda i: (0, 0)),
            pl.BlockSpec((1, m), lambda i: (0, 0)),
        ],
        out_specs=pl.BlockSpec((BR, m), lambda i: (i, 0)),
        out_shape=jax.ShapeDtypeStruct((n, m), jnp.float32),
    )(h, agg, deg, w_self, b)


def kernel(x, edge_index, W_self0, W_neigh0, b0,
           W_self1, W_neigh1, b1, W_self2, W_neigh2, b2):
    xp = jnp.pad(x, ((0, NPAD - N_NODES), (0, 0)))
    src = edge_index[0]
    dst = edge_index[1]
    npadrows = NPAD - N_NODES
    pad_f = N_NODES + (jnp.arange(EPT_PAD - EPT, dtype=jnp.int32) % npadrows)
    pad_f = jnp.broadcast_to(pad_f, (NS, EPT_PAD - EPT))
    pad_e = N_NODES + (jnp.arange(EPW_PAD - EPW, dtype=jnp.int32) % npadrows)
    pad_e = jnp.broadcast_to(pad_e, (NW, EPW_PAD - EPW))
    src_f = jnp.pad(src.reshape(NS, EPT), ((0, 0), (0, EPT_PAD - EPT)))
    src_f = src_f.reshape(NS, NCH_F, CHUNK)
    dst_f = jnp.concatenate([dst.reshape(NS, EPT), pad_f], axis=1)
    dst_f = dst_f.reshape(NS, NCH_F, CHUNK)
    src_e = jnp.pad(src.reshape(NW, EPW), ((0, 0), (0, EPW_PAD - EPW)))
    src_e = src_e.reshape(NW, NCH_E, CHUNK)
    dst_e = jnp.concatenate([dst.reshape(NW, EPW), pad_e], axis=1)
    dst_e = dst_e.reshape(NW, NCH_E, CHUNK)
    zeros64 = jnp.zeros((NPAD, DH), jnp.float32)
    zeros16 = jnp.zeros((NPAD, 16), jnp.float32)
    ones16 = jnp.ones((CHUNK, 16), jnp.float32)
    b0r = b0.reshape(1, -1)
    b1r = b1.reshape(1, -1)
    b2r = b2.reshape(1, -1)

    p0 = _matmul_split(xp, W_neigh0)
    agg0, deg = _sc_agg_split_deg(p0, src_f, dst_f, zeros64, zeros16, ones16)
    h1, p1 = _mid_layer(xp, agg0, deg, W_self0, b0r, W_neigh1, split_p=True)
    agg1 = _sc_agg_split(p1, src_f, dst_f, zeros64)
    h2, p2 = _mid_layer(h1, agg1, deg, W_self1, b1r, W_neigh2, split_p=False)
    agg2 = _sc_agg16(p2, src_e, dst_e, zeros16)
    out = _last_layer(h2, agg2, deg, W_self2, b2r)
    return out[:N_NODES]


# speedup vs baseline: 1.0082x; 1.0082x over previous
import jax
import jax.numpy as jnp
from jax import lax
from jax.experimental import pallas as pl
from jax.experimental.pallas import tpu as pltpu
from jax.experimental.pallas import tpu_sc as plsc

N_NODES = 10000
NPAD = 10240
N_EDGES = 320000
D = 128
DH = D // 2
C = 16

NC = 2
NS = 16
NW = NC * NS
CHUNK = 128

EPT = N_EDGES // NS
NCH_F = 158
EPT_PAD = NCH_F * CHUNK

EPW = N_EDGES // NW
NCH_E = 80
EPW_PAD = NCH_E * CHUNK

RPT = NPAD // NS

BR = 5120

_MESH = plsc.VectorSubcoreMesh(core_axis_name="c", subcore_axis_name="s",
                               num_cores=NC, num_subcores=NS)


NBUF = 4


def _edge_loop_sync(table, src_v, dst_v, acc, bufs, gsem, nchunk, deg=None,
                    deg_parity=0):
    def chunk(j, b):
        pltpu.make_async_copy(table.at[src_v.at[j]], bufs[b], gsem[b]).wait()
        pltpu.sync_copy(bufs[b], acc.at[dst_v.at[j]], add=True)
        if deg is not None and b % 2 == deg_parity:
            pltpu.sync_copy(deg[0], deg[1].at[dst_v.at[j]], add=True)

        @pl.when(j + 2 < nchunk)
        def _():
            pltpu.make_async_copy(table.at[src_v.at[j + 2]], bufs[b],
                                  gsem[b]).start()

    pltpu.make_async_copy(table.at[src_v.at[0]], bufs[0], gsem[0]).start()
    pltpu.make_async_copy(table.at[src_v.at[1]], bufs[1], gsem[1]).start()

    def step(k, carry):
        chunk(2 * k, 0)
        chunk(2 * k + 1, 1)
        return carry

    lax.fori_loop(0, nchunk // 2, step, 0)


def _edge_loop(table, src_v, dst_v, acc, bufs, gsem, ssem, nchunk, deg=None):
    pltpu.make_async_copy(table.at[src_v.at[0]], bufs[0], gsem[0]).start()
    pltpu.make_async_copy(table.at[src_v.at[1]], bufs[1], gsem[1]).start()

    def step(k, carry):
        for b in range(NBUF):
            j = NBUF * k + b
            bn = (b + 2) % NBUF
            pltpu.make_async_copy(table.at[src_v.at[j]], bufs[b], gsem[b]).wait()
            pltpu.async_copy(bufs[b], acc.at[dst_v.at[j]], ssem[b], add=True)
            if deg is not None:
                pltpu.async_copy(deg[0], deg[1].at[dst_v.at[j]], deg[2][b],
                                 add=True)

            @pl.when(j >= 2)
            def _():
                pltpu.make_async_copy(bufs[bn], acc.at[dst_v.at[j - 2]],
                                      ssem[bn]).wait()
                if deg is not None:
                    pltpu.make_async_copy(deg[0], deg[1].at[dst_v.at[j - 2]],
                                          deg[2][bn]).wait()

            @pl.when(j + 2 < nchunk)
            def _():
                pltpu.make_async_copy(table.at[src_v.at[j + 2]], bufs[bn],
                                      gsem[bn]).start()
        return carry

    lax.fori_loop(0, nchunk // NBUF, step, 0)
    for b in (2, 3):
        j = nchunk - NBUF + b
        pltpu.make_async_copy(bufs[b], acc.at[dst_v.at[j]], ssem[b]).wait()
        if deg is not None:
            pltpu.make_async_copy(deg[0], deg[1].at[dst_v.at[j]],
                                  deg[2][b]).wait()


def _make_sc_agg_split(with_deg):
    out_type = [jax.ShapeDtypeStruct((NC, NPAD, DH), jnp.float32)]
    scratch = [
        pltpu.VMEM((NCH_F, CHUNK), jnp.int32),
        pltpu.VMEM((NCH_F, CHUNK), jnp.int32),
        pltpu.VMEM((CHUNK, DH), jnp.float32),
        pltpu.VMEM((CHUNK, DH), jnp.float32),
        pltpu.VMEM_SHARED((NPAD, DH), jnp.float32),
        pltpu.SemaphoreType.DMA,
        pltpu.SemaphoreType.DMA,
    ]
    if with_deg:
        out_type.append(jax.ShapeDtypeStruct((NC, NPAD, 16), jnp.float32))
        scratch += [
            pltpu.VMEM((CHUNK, 16), jnp.float32),
            pltpu.VMEM_SHARED((NPAD, 16), jnp.float32),
        ]

    def body(p, srcs, dsts, zeros_d, *rest):
        if with_deg:
            (zeros16, ones_h, agg_out, deg_out,
             src_v, dst_v, buf0, buf1, acc, sem0, sem1, ones_v, dacc) = rest
        else:
            agg_out, src_v, dst_v, buf0, buf1, acc, sem0, sem1 = rest
        bufs = (buf0, buf1)
        gsem = (sem0, sem1)
        c = lax.axis_index("c")
        s = lax.axis_index("s")
        pltpu.sync_copy(srcs.at[s], src_v)
        pltpu.sync_copy(dsts.at[s], dst_v)
        pltpu.sync_copy(zeros_d.at[pl.ds(s * RPT, RPT)],
                        acc.at[pl.ds(s * RPT, RPT)])
        if with_deg:
            pltpu.sync_copy(zeros16.at[pl.ds(s * RPT, RPT)],
                            dacc.at[pl.ds(s * RPT, RPT)])
            pltpu.sync_copy(ones_h, ones_v)
        plsc.subcore_barrier()

        deg_args = (ones_v, dacc) if with_deg else None

        @pl.when(c == 0)
        def _():
            _edge_loop_sync(p.at[0], src_v, dst_v, acc, bufs, gsem, NCH_F,
                            deg=deg_args, deg_parity=0)

        @pl.when(c == 1)
        def _():
            _edge_loop_sync(p.at[1], src_v, dst_v, acc, bufs, gsem, NCH_F,
                            deg=deg_args, deg_parity=1)

        plsc.subcore_barrier()
        pltpu.sync_copy(acc.at[pl.ds(s * RPT, RPT)],
                        agg_out.at[c, pl.ds(s * RPT, RPT)])
        if with_deg:
            pltpu.sync_copy(dacc.at[pl.ds(s * RPT, RPT)],
                            deg_out.at[c, pl.ds(s * RPT, RPT)])

    out_spec = tuple(out_type) if len(out_type) > 1 else out_type[0]
    return pl.kernel(body, out_type=out_spec, mesh=_MESH,
                     scratch_types=tuple(scratch),
                     compiler_params=pltpu.CompilerParams(
                         use_tc_tiling_on_sc=False))


def _make_sc_agg_edges(d):
    scratch = [
        pltpu.VMEM((NCH_E, CHUNK), jnp.int32),
        pltpu.VMEM((NCH_E, CHUNK), jnp.int32),
        pltpu.VMEM((CHUNK, d), jnp.float32),
        pltpu.VMEM((CHUNK, d), jnp.float32),
        pltpu.VMEM_SHARED((NPAD, d), jnp.float32),
        pltpu.SemaphoreType.DMA,
        pltpu.SemaphoreType.DMA,
    ]

    def body(p, srcs, dsts, zeros_d, agg_out,
             src_v, dst_v, buf0, buf1, acc, sem0, sem1):
        c = lax.axis_index("c")
        s = lax.axis_index("s")
        w = c * NS + s
        pltpu.sync_copy(srcs.at[w], src_v)
        pltpu.sync_copy(dsts.at[w], dst_v)
        pltpu.sync_copy(zeros_d.at[pl.ds(s * RPT, RPT)],
                        acc.at[pl.ds(s * RPT, RPT)])
        plsc.subcore_barrier()
        _edge_loop_sync(p, src_v, dst_v, acc, (buf0, buf1), (sem0, sem1),
                        NCH_E)
        plsc.subcore_barrier()
        pltpu.sync_copy(acc.at[pl.ds(s * RPT, RPT)],
                        agg_out.at[c, pl.ds(s * RPT, RPT)])

    return pl.kernel(body,
                     out_type=jax.ShapeDtypeStruct((NC, NPAD, d), jnp.float32),
                     mesh=_MESH, scratch_types=tuple(scratch),
                     compiler_params=pltpu.CompilerParams(
                         use_tc_tiling_on_sc=False))


_sc_agg_split_deg = _make_sc_agg_split(with_deg=True)
_sc_agg_split = _make_sc_agg_split(with_deg=False)
_sc_agg16 = _make_sc_agg_edges(C)


def _mm_split_body(x_ref, w_ref, o_ref):
    p = jnp.dot(x_ref[...], w_ref[...], preferred_element_type=jnp.float32)
    o_ref[0] = p[:, :DH]
    o_ref[1] = p[:, DH:]


def _matmul_split(x, w):
    n, k = x.shape
    return pl.pallas_call(
        _mm_split_body,
        grid=(n // BR,),
        in_specs=[pl.BlockSpec((BR, k), lambda i: (i, 0)),
                  pl.BlockSpec((k, D), lambda i: (0, 0))],
        out_specs=pl.BlockSpec((NC, BR, DH), lambda i: (0, i, 0)),
        out_shape=jax.ShapeDtypeStruct((NC, n, DH), jnp.float32),
    )(x, w)


def _mid_body(split_p, x_ref, agg_ref, deg_ref, ws_ref, b_ref, wn_ref,
              h_ref, p_ref):
    agg = jnp.concatenate([agg_ref[0], agg_ref[1]], axis=1)
    deg = deg_ref[0, :, 0:1] + deg_ref[1, :, 0:1]
    inv = 1.0 / jnp.maximum(deg, 1.0)
    z = (jnp.dot(x_ref[...], ws_ref[...], preferred_element_type=jnp.float32)
         + agg * inv + b_ref[...])
    alpha = 1.6732632423543772
    scale = 1.0507009873554805
    h = scale * jnp.where(z > 0, z, alpha * (jnp.exp(z) - 1.0))
    h_ref[...] = h
    p = jnp.dot(h, wn_ref[...], preferred_element_type=jnp.float32)
    if split_p:
        p_ref[0] = p[:, :DH]
        p_ref[1] = p[:, DH:]
    else:
        p_ref[...] = p


def _mid_layer(x, agg, deg, w_self, b, w_neigh_next, split_p):
    n, k = x.shape
    m = w_neigh_next.shape[1]
    if split_p:
        p_spec = pl.BlockSpec((NC, BR, DH), lambda i: (0, i, 0))
        p_shape = jax.ShapeDtypeStruct((NC, n, DH), jnp.float32)
    else:
        p_spec = pl.BlockSpec((BR, m), lambda i: (i, 0))
        p_shape = jax.ShapeDtypeStruct((n, m), jnp.float32)
    body = lambda *refs: _mid_body(split_p, *refs)
    return pl.pallas_call(
        body,
        grid=(n // BR,),
        in_specs=[
            pl.BlockSpec((BR, k), lambda i: (i, 0)),
            pl.BlockSpec((NC, BR, DH), lambda i: (0, i, 0)),
            pl.BlockSpec((NC, BR, 16), lambda i: (0, i, 0)),
            pl.BlockSpec((k, k), lambda i: (0, 0)),
            pl.BlockSpec((1, k), lambda i: (0, 0)),
            pl.BlockSpec((k, m), lambda i: (0, 0)),
        ],
        out_specs=[pl.BlockSpec((BR, k), lambda i: (i, 0)), p_spec],
        out_shape=[jax.ShapeDtypeStruct((n, k), jnp.float32), p_shape],
    )(x, agg, deg, w_self, b, w_neigh_next)


def _last_body(h_ref, agg_ref, deg_ref, ws_ref, b_ref, o_ref):
    agg = agg_ref[0] + agg_ref[1]
    deg = deg_ref[0, :, 0:1] + deg_ref[1, :, 0:1]
    inv = 1.0 / jnp.maximum(deg, 1.0)
    z = (jnp.dot(h_ref[...], ws_ref[...], preferred_element_type=jnp.float32)
         + agg * inv + b_ref[...])
    z = z - jnp.max(z, axis=1, keepdims=True)
    e = jnp.exp(z)
    o_ref[...] = e / jnp.sum(e, axis=1, keepdims=True)


def _last_layer(h, agg, deg, w_self, b):
    n, k = h.shape
    m = w_self.shape[1]
    return pl.pallas_call(
        _last_body,
        grid=(n // BR,),
        in_specs=[
            pl.BlockSpec((BR, k), lambda i: (i, 0)),
            pl.BlockSpec((NC, BR, m), lambda i: (0, i, 0)),
            pl.BlockSpec((NC, BR, 16), lambda i: (0, i, 0)),
            pl.BlockSpec((k, m), lambda i: (0, 0)),
            pl.BlockSpec((1, m), lambda i: (0, 0)),
        ],
        out_specs=pl.BlockSpec((BR, m), lambda i: (i, 0)),
        out_shape=jax.ShapeDtypeStruct((n, m), jnp.float32),
    )(h, agg, deg, w_self, b)


def kernel(x, edge_index, W_self0, W_neigh0, b0,
           W_self1, W_neigh1, b1, W_self2, W_neigh2, b2):
    xp = jnp.pad(x, ((0, NPAD - N_NODES), (0, 0)))
    src = edge_index[0]
    dst = edge_index[1]
    npadrows = NPAD - N_NODES
    pad_f = N_NODES + (jnp.arange(EPT_PAD - EPT, dtype=jnp.int32) % npadrows)
    pad_f = jnp.broadcast_to(pad_f, (NS, EPT_PAD - EPT))
    pad_e = N_NODES + (jnp.arange(EPW_PAD - EPW, dtype=jnp.int32) % npadrows)
    pad_e = jnp.broadcast_to(pad_e, (NW, EPW_PAD - EPW))
    src_f = jnp.pad(src.reshape(NS, EPT), ((0, 0), (0, EPT_PAD - EPT)))
    src_f = src_f.reshape(NS, NCH_F, CHUNK)
    dst_f = jnp.concatenate([dst.reshape(NS, EPT), pad_f], axis=1)
    dst_f = dst_f.reshape(NS, NCH_F, CHUNK)
    src_e = jnp.pad(src.reshape(NW, EPW), ((0, 0), (0, EPW_PAD - EPW)))
    src_e = src_e.reshape(NW, NCH_E, CHUNK)
    dst_e = jnp.concatenate([dst.reshape(NW, EPW), pad_e], axis=1)
    dst_e = dst_e.reshape(NW, NCH_E, CHUNK)
    zeros64 = jnp.zeros((NPAD, DH), jnp.float32)
    zeros16 = jnp.zeros((NPAD, 16), jnp.float32)
    ones16 = jnp.ones((CHUNK, 16), jnp.float32)
    b0r = b0.reshape(1, -1)
    b1r = b1.reshape(1, -1)
    b2r = b2.reshape(1, -1)

    p0 = _matmul_split(xp, W_neigh0)
    agg0, deg = _sc_agg_split_deg(p0, src_f, dst_f, zeros64, zeros16, ones16)
    h1, p1 = _mid_layer(xp, agg0, deg, W_self0, b0r, W_neigh1, split_p=True)
    agg1 = _sc_agg_split(p1, src_f, dst_f, zeros64)
    h2, p2 = _mid_layer(h1, agg1, deg, W_self1, b1r, W_neigh2, split_p=False)
    agg2 = _sc_agg16(p2, src_e, dst_e, zeros16)
    out = _last_layer(h2, agg2, deg, W_self2, b2r)
    return out[:N_NODES]


# speedup vs baseline: 1.0087x; 1.0005x over previous
import jax
import jax.numpy as jnp
from jax import lax
from jax.experimental import pallas as pl
from jax.experimental.pallas import tpu as pltpu
from jax.experimental.pallas import tpu_sc as plsc

N_NODES = 10000
NPAD = 10240
N_EDGES = 320000
D = 128
DH = D // 2
C = 16

NC = 2
NS = 16
NW = NC * NS
CHUNK = 128

EPT = N_EDGES // NS
NCH_F = 158
EPT_PAD = NCH_F * CHUNK

EPW = N_EDGES // NW
NCH_E = 80
EPW_PAD = NCH_E * CHUNK

RPT = NPAD // NS

BR = 5120

_MESH = plsc.VectorSubcoreMesh(core_axis_name="c", subcore_axis_name="s",
                               num_cores=NC, num_subcores=NS)


def _edge_loop_sync(table, src_v, dst_v, acc, bufs, gsem, nchunk, deg=None,
                    deg_parity=0):
    def chunk(j, b):
        pltpu.make_async_copy(table.at[src_v.at[j]], bufs[b], gsem[b]).wait()
        pltpu.sync_copy(bufs[b], acc.at[dst_v.at[j]], add=True)
        if deg is not None and b % 2 == deg_parity:
            pltpu.sync_copy(deg[0], deg[1].at[dst_v.at[j]], add=True)

        @pl.when(j + 2 < nchunk)
        def _():
            pltpu.make_async_copy(table.at[src_v.at[j + 2]], bufs[b],
                                  gsem[b]).start()

    pltpu.make_async_copy(table.at[src_v.at[0]], bufs[0], gsem[0]).start()
    pltpu.make_async_copy(table.at[src_v.at[1]], bufs[1], gsem[1]).start()

    def step(k, carry):
        chunk(2 * k, 0)
        chunk(2 * k + 1, 1)
        return carry

    lax.fori_loop(0, nchunk // 2, step, 0)


def _make_sc_agg_split(with_deg):
    out_type = [jax.ShapeDtypeStruct((NC, NPAD, DH), jnp.float32)]
    scratch = [
        pltpu.VMEM((NCH_F, CHUNK), jnp.int32),
        pltpu.VMEM((NCH_F, CHUNK), jnp.int32),
        pltpu.VMEM((CHUNK, DH), jnp.float32),
        pltpu.VMEM((CHUNK, DH), jnp.float32),
        pltpu.VMEM_SHARED((NPAD, DH), jnp.float32),
        pltpu.SemaphoreType.DMA,
        pltpu.SemaphoreType.DMA,
    ]
    if with_deg:
        out_type.append(jax.ShapeDtypeStruct((NC, NPAD, 16), jnp.float32))
        scratch += [
            pltpu.VMEM((CHUNK, 16), jnp.float32),
            pltpu.VMEM_SHARED((NPAD, 16), jnp.float32),
        ]

    def body(p, srcs, dsts, zeros_d, *rest):
        if with_deg:
            (zeros16, ones_h, agg_out, deg_out,
             src_v, dst_v, buf0, buf1, acc, sem0, sem1, ones_v, dacc) = rest
        else:
            agg_out, src_v, dst_v, buf0, buf1, acc, sem0, sem1 = rest
        bufs = (buf0, buf1)
        gsem = (sem0, sem1)
        c = lax.axis_index("c")
        s = lax.axis_index("s")
        pltpu.sync_copy(srcs.at[s], src_v)
        pltpu.sync_copy(dsts.at[s], dst_v)
        pltpu.sync_copy(zeros_d.at[pl.ds(s * RPT, RPT)],
                        acc.at[pl.ds(s * RPT, RPT)])
        if with_deg:
            pltpu.sync_copy(zeros16.at[pl.ds(s * RPT, RPT)],
                            dacc.at[pl.ds(s * RPT, RPT)])
            pltpu.sync_copy(ones_h, ones_v)
        plsc.subcore_barrier()

        deg_args = (ones_v, dacc) if with_deg else None

        @pl.when(c == 0)
        def _():
            _edge_loop_sync(p.at[0], src_v, dst_v, acc, bufs, gsem, NCH_F,
                            deg=deg_args, deg_parity=0)

        @pl.when(c == 1)
        def _():
            _edge_loop_sync(p.at[1], src_v, dst_v, acc, bufs, gsem, NCH_F,
                            deg=deg_args, deg_parity=1)

        plsc.subcore_barrier()
        pltpu.sync_copy(acc.at[pl.ds(s * RPT, RPT)],
                        agg_out.at[c, pl.ds(s * RPT, RPT)])
        if with_deg:
            pltpu.sync_copy(dacc.at[pl.ds(s * RPT, RPT)],
                            deg_out.at[c, pl.ds(s * RPT, RPT)])

    out_spec = tuple(out_type) if len(out_type) > 1 else out_type[0]
    return pl.kernel(body, out_type=out_spec, mesh=_MESH,
                     scratch_types=tuple(scratch),
                     compiler_params=pltpu.CompilerParams(
                         use_tc_tiling_on_sc=False))


def _make_sc_agg_edges(d):
    scratch = [
        pltpu.VMEM((NCH_E, CHUNK), jnp.int32),
        pltpu.VMEM((NCH_E, CHUNK), jnp.int32),
        pltpu.VMEM((CHUNK, d), jnp.float32),
        pltpu.VMEM((CHUNK, d), jnp.float32),
        pltpu.VMEM_SHARED((NPAD, d), jnp.float32),
        pltpu.SemaphoreType.DMA,
        pltpu.SemaphoreType.DMA,
    ]

    def body(p, srcs, dsts, zeros_d, agg_out,
             src_v, dst_v, buf0, buf1, acc, sem0, sem1):
        c = lax.axis_index("c")
        s = lax.axis_index("s")
        w = c * NS + s
        pltpu.sync_copy(srcs.at[w], src_v)
        pltpu.sync_copy(dsts.at[w], dst_v)
        pltpu.sync_copy(zeros_d.at[pl.ds(s * RPT, RPT)],
                        acc.at[pl.ds(s * RPT, RPT)])
        plsc.subcore_barrier()
        _edge_loop_sync(p, src_v, dst_v, acc, (buf0, buf1), (sem0, sem1),
                        NCH_E)
        plsc.subcore_barrier()
        pltpu.sync_copy(acc.at[pl.ds(s * RPT, RPT)],
                        agg_out.at[c, pl.ds(s * RPT, RPT)])

    return pl.kernel(body,
                     out_type=jax.ShapeDtypeStruct((NC, NPAD, d), jnp.float32),
                     mesh=_MESH, scratch_types=tuple(scratch),
                     compiler_params=pltpu.CompilerParams(
                         use_tc_tiling_on_sc=False))


_sc_agg_split_deg = _make_sc_agg_split(with_deg=True)
_sc_agg_split = _make_sc_agg_split(with_deg=False)
_sc_agg16 = _make_sc_agg_edges(C)


def _mm_split_body(x_ref, w_ref, o_ref):
    p = jnp.dot(x_ref[...], w_ref[...], preferred_element_type=jnp.float32)
    o_ref[0] = p[:, :DH]
    o_ref[1] = p[:, DH:]


def _matmul_split(x, w):
    n, k = x.shape
    return pl.pallas_call(
        _mm_split_body,
        grid=(n // BR,),
        in_specs=[pl.BlockSpec((BR, k), lambda i: (i, 0)),
                  pl.BlockSpec((k, D), lambda i: (0, 0))],
        out_specs=pl.BlockSpec((NC, BR, DH), lambda i: (0, i, 0)),
        out_shape=jax.ShapeDtypeStruct((NC, n, DH), jnp.float32),
    )(x, w)


def _mid_body(split_p, x_ref, agg_ref, deg_ref, ws_ref, b_ref, wn_ref,
              h_ref, p_ref):
    agg = jnp.concatenate([agg_ref[0], agg_ref[1]], axis=1)
    deg = deg_ref[0, :, 0:1] + deg_ref[1, :, 0:1]
    inv = 1.0 / jnp.maximum(deg, 1.0)
    z = (jnp.dot(x_ref[...], ws_ref[...], preferred_element_type=jnp.float32)
         + agg * inv + b_ref[...])
    alpha = 1.6732632423543772
    scale = 1.0507009873554805
    h = scale * jnp.where(z > 0, z, alpha * (jnp.exp(z) - 1.0))
    h_ref[...] = h
    p = jnp.dot(h, wn_ref[...], preferred_element_type=jnp.float32)
    if split_p:
        p_ref[0] = p[:, :DH]
        p_ref[1] = p[:, DH:]
    else:
        p_ref[...] = p


def _mid_layer(x, agg, deg, w_self, b, w_neigh_next, split_p):
    n, k = x.shape
    m = w_neigh_next.shape[1]
    if split_p:
        p_spec = pl.BlockSpec((NC, BR, DH), lambda i: (0, i, 0))
        p_shape = jax.ShapeDtypeStruct((NC, n, DH), jnp.float32)
    else:
        p_spec = pl.BlockSpec((BR, m), lambda i: (i, 0))
        p_shape = jax.ShapeDtypeStruct((n, m), jnp.float32)
    body = lambda *refs: _mid_body(split_p, *refs)
    return pl.pallas_call(
        body,
        grid=(n // BR,),
        in_specs=[
            pl.BlockSpec((BR, k), lambda i: (i, 0)),
            pl.BlockSpec((NC, BR, DH), lambda i: (0, i, 0)),
            pl.BlockSpec((NC, BR, 16), lambda i: (0, i, 0)),
            pl.BlockSpec((k, k), lambda i: (0, 0)),
            pl.BlockSpec((1, k), lambda i: (0, 0)),
            pl.BlockSpec((k, m), lambda i: (0, 0)),
        ],
        out_specs=[pl.BlockSpec((BR, k), lambda i: (i, 0)), p_spec],
        out_shape=[jax.ShapeDtypeStruct((n, k), jnp.float32), p_shape],
    )(x, agg, deg, w_self, b, w_neigh_next)


def _last_body(h_ref, agg_ref, deg_ref, ws_ref, b_ref, o_ref):
    agg = agg_ref[0] + agg_ref[1]
    deg = deg_ref[0, :, 0:1] + deg_ref[1, :, 0:1]
    inv = 1.0 / jnp.maximum(deg, 1.0)
    z = (jnp.dot(h_ref[...], ws_ref[...], preferred_element_type=jnp.float32)
         + agg * inv + b_ref[...])
    z = z - jnp.max(z, axis=1, keepdims=True)
    e = jnp.exp(z)
    o_ref[...] = e / jnp.sum(e, axis=1, keepdims=True)


def _last_layer(h, agg, deg, w_self, b):
    n, k = h.shape
    m = w_self.shape[1]
    return pl.pallas_call(
        _last_body,
        grid=(n // BR,),
        in_specs=[
            pl.BlockSpec((BR, k), lambda i: (i, 0)),
            pl.BlockSpec((NC, BR, m), lambda i: (0, i, 0)),
            pl.BlockSpec((NC, BR, 16), lambda i: (0, i, 0)),
            pl.BlockSpec((k, m), lambda i: (0, 0)),
            pl.BlockSpec((1, m), lambda i: (0, 0)),
        ],
        out_specs=pl.BlockSpec((BR, m), lambda i: (i, 0)),
        out_shape=jax.ShapeDtypeStruct((n, m), jnp.float32),
    )(h, agg, deg, w_self, b)


def kernel(x, edge_index, W_self0, W_neigh0, b0,
           W_self1, W_neigh1, b1, W_self2, W_neigh2, b2):
    xp = jnp.pad(x, ((0, NPAD - N_NODES), (0, 0)))
    src = edge_index[0]
    dst = edge_index[1]
    npadrows = NPAD - N_NODES
    pad_f = N_NODES + (jnp.arange(EPT_PAD - EPT, dtype=jnp.int32) % npadrows)
    pad_f = jnp.broadcast_to(pad_f, (NS, EPT_PAD - EPT))
    pad_e = N_NODES + (jnp.arange(EPW_PAD - EPW, dtype=jnp.int32) % npadrows)
    pad_e = jnp.broadcast_to(pad_e, (NW, EPW_PAD - EPW))
    src_f = jnp.pad(src.reshape(NS, EPT), ((0, 0), (0, EPT_PAD - EPT)))
    src_f = src_f.reshape(NS, NCH_F, CHUNK)
    dst_f = jnp.concatenate([dst.reshape(NS, EPT), pad_f], axis=1)
    dst_f = dst_f.reshape(NS, NCH_F, CHUNK)
    src_e = jnp.pad(src.reshape(NW, EPW), ((0, 0), (0, EPW_PAD - EPW)))
    src_e = src_e.reshape(NW, NCH_E, CHUNK)
    dst_e = jnp.concatenate([dst.reshape(NW, EPW), pad_e], axis=1)
    dst_e = dst_e.reshape(NW, NCH_E, CHUNK)
    zeros64 = jnp.zeros((NPAD, DH), jnp.float32)
    zeros16 = jnp.zeros((NPAD, 16), jnp.float32)
    ones16 = jnp.ones((CHUNK, 16), jnp.float32)
    b0r = b0.reshape(1, -1)
    b1r = b1.reshape(1, -1)
    b2r = b2.reshape(1, -1)

    p0 = _matmul_split(xp, W_neigh0)
    agg0, deg = _sc_agg_split_deg(p0, src_f, dst_f, zeros64, zeros16, ones16)
    h1, p1 = _mid_layer(xp, agg0, deg, W_self0, b0r, W_neigh1, split_p=True)
    agg1 = _sc_agg_split(p1, src_f, dst_f, zeros64)
    h2, p2 = _mid_layer(h1, agg1, deg, W_self1, b1r, W_neigh2, split_p=False)
    agg2 = _sc_agg16(p2, src_e, dst_e, zeros16)
    out = _last_layer(h2, agg2, deg, W_self2, b2r)
    return out[:N_NODES]
